# baseline (speedup 1.0000x reference)
import functools

import numpy as np
import jax
import jax.numpy as jnp
from jax import lax
from jax.experimental import pallas as pl
from jax.experimental.pallas import tpu as pltpu
from jax.experimental.pallas import tpu_sc as plsc

_OC, _IC, _ORD, _K = 384, 192, 4, 5
_ROW = _ORD * _K * _K
_QUAD = 4 * _ROW
_NQUAD = _IC // 4
_BLK = _IC * _ROW
_NW = 32
_OC_PER_W = _OC // _NW
_LANES = 16
_VPQ = _QUAD // _LANES


def _perm_tables() -> np.ndarray:
    a = np.arange(_ROW).reshape(_ORD, _K, _K)
    tabs = []
    for r in (1, 2, 3):
        p = np.rot90(np.roll(a, shift=r, axis=0), k=r, axes=(-2, -1)).reshape(_ROW)
        tabs.append(np.concatenate([p + q * _ROW for q in range(4)]))
    return np.concatenate(tabs).astype(np.int32)


_IDX_TAB = _perm_tables()

_MESH = plsc.VectorSubcoreMesh(core_axis_name="c", subcore_axis_name="s",
                               num_cores=2, num_subcores=16)


@functools.partial(
    pl.kernel,
    out_type=jax.ShapeDtypeStruct((_OC, _ORD, _BLK), jnp.float32),
    mesh=_MESH,
    scratch_types=[
        pltpu.VMEM((3 * _QUAD,), jnp.int32),
        pltpu.VMEM((_BLK,), jnp.float32),
        pltpu.VMEM((_BLK,), jnp.float32),
    ],
    compiler_params=pltpu.CompilerParams(needs_layout_passes=False),
)
def _bank(w_hbm, idx_hbm, out_hbm, idx_v, in_v, out_v):
    wid = lax.axis_index("s") * 2 + lax.axis_index("c")
    pltpu.sync_copy(idx_hbm, idx_v)

    def per_oc(t, carry):
        oc = wid * _OC_PER_W + t
        pltpu.sync_copy(w_hbm.at[oc], in_v)
        pltpu.sync_copy(in_v, out_hbm.at[oc, 0])
        for r in range(3):
            def per_quad(q, c):
                base = jnp.full((_LANES,), q * _QUAD, jnp.int32)
                off = q * _QUAD
                for v in range(_VPQ):
                    idx = idx_v[pl.ds(r * _QUAD + v * _LANES, _LANES)] + base
                    vals = plsc.load_gather(in_v, [idx])
                    out_v[pl.ds(off + v * _LANES, _LANES)] = vals
                return c
            lax.fori_loop(0, _NQUAD, per_quad, 0)
            pltpu.sync_copy(out_v, out_hbm.at[oc, r + 1])
        return carry

    lax.fori_loop(0, _OC_PER_W, per_oc, 0)


def kernel(weight):
    w2 = weight.reshape(_OC, _BLK)
    out = _bank(w2, jnp.asarray(_IDX_TAB))
    return out.reshape(_OC, _ORD, _IC, _ORD, _K, _K)


# speedup vs baseline: 1.7579x; 1.7579x over previous
import functools

import numpy as np
import jax
import jax.numpy as jnp
from jax import lax
from jax.experimental import pallas as pl
from jax.experimental.pallas import tpu as pltpu
from jax.experimental.pallas import tpu_sc as plsc

_OC, _IC, _ORD, _K = 384, 192, 4, 5
_ROW = _ORD * _K * _K
_BLK = _IC * _ROW
_NW = 32
_OC_PER_W = _OC // _NW
_LANES = 16


def _perm_tables() -> np.ndarray:
    a = np.arange(_ROW).reshape(_ORD, _K, _K)
    i = np.arange(_BLK)
    tabs = []
    for r in (1, 2, 3):
        p = np.rot90(np.roll(a, shift=r, axis=0), k=r, axes=(-2, -1)).reshape(_ROW)
        tabs.append((i // _ROW) * _ROW + p[i % _ROW])
    return np.concatenate(tabs).astype(np.int32)


_IDX_TAB = _perm_tables()

_MESH = plsc.VectorSubcoreMesh(core_axis_name="c", subcore_axis_name="s",
                               num_cores=2, num_subcores=16)


@functools.partial(
    pl.kernel,
    out_type=jax.ShapeDtypeStruct((_OC, _ORD, _BLK), jnp.float32),
    mesh=_MESH,
    scratch_types=[
        pltpu.VMEM((3 * _BLK,), jnp.int32),
        pltpu.VMEM((_BLK,), jnp.float32),
        pltpu.VMEM((2, _BLK), jnp.float32),
        pltpu.SemaphoreType.DMA,
        pltpu.SemaphoreType.DMA,
        pltpu.SemaphoreType.DMA,
    ],
    compiler_params=pltpu.CompilerParams(needs_layout_passes=False),
)
def _bank(w_hbm, idx_hbm, out_hbm, idx_v, in_v, out_v, sem0, sem_a, sem_b):
    wid = lax.axis_index("s") * 2 + lax.axis_index("c")
    pltpu.sync_copy(idx_hbm, idx_v)
    out_sems = (sem_a, sem_b)

    def per_oc(t, carry):
        oc = wid * _OC_PER_W + t
        pltpu.sync_copy(w_hbm.at[oc], in_v)
        c0 = pltpu.async_copy(in_v, out_hbm.at[oc, 0], sem0)
        copies = []
        for r in range(3):
            b = r & 1

            def body(i, r=r, b=b):
                idx = idx_v[pl.ds(r * _BLK + i, _LANES)]
                vals = plsc.load_gather(in_v, [idx])
                out_v[b, pl.ds(i, _LANES)] = vals

            if r == 2:
                copies[0].wait()
            plsc.parallel_loop(0, _BLK, step=_LANES, unroll=8)(body)
            copies.append(
                pltpu.async_copy(out_v.at[b], out_hbm.at[oc, r + 1], out_sems[b]))
        c0.wait()
        copies[1].wait()
        copies[2].wait()
        return carry

    lax.fori_loop(0, _OC_PER_W, per_oc, 0)


def kernel(weight):
    w2 = weight.reshape(_OC, _BLK)
    out = _bank(w2, jnp.asarray(_IDX_TAB))
    return out.reshape(_OC, _ORD, _IC, _ORD, _K, _K)


# speedup vs baseline: 1.8095x; 1.0294x over previous
import functools

import numpy as np
import jax
import jax.numpy as jnp
from jax import lax
from jax.experimental import pallas as pl
from jax.experimental.pallas import tpu as pltpu
from jax.experimental.pallas import tpu_sc as plsc

_OC, _IC, _ORD, _K = 384, 192, 4, 5
_ROW = _ORD * _K * _K
_BLK = _IC * _ROW
_NW = 32
_OC_PER_W = _OC // _NW
_LANES = 16


_QUAD = 4 * _ROW
_VPQ = _QUAD // _LANES
_NQUAD = _BLK // _QUAD


def _perm_tables() -> np.ndarray:
    a = np.arange(_ROW).reshape(_ORD, _K, _K)
    tabs = []
    for r in (1, 2, 3):
        p = np.rot90(np.roll(a, shift=r, axis=0), k=r, axes=(-2, -1)).reshape(_ROW)
        tabs.append(np.concatenate([p + q * _ROW for q in range(4)]))
    return np.concatenate(tabs).astype(np.int32)


_IDX_TAB = _perm_tables()

_MESH = plsc.VectorSubcoreMesh(core_axis_name="c", subcore_axis_name="s",
                               num_cores=2, num_subcores=16)


@functools.partial(
    pl.kernel,
    out_type=jax.ShapeDtypeStruct((_OC, _ORD, _BLK), jnp.float32),
    mesh=_MESH,
    scratch_types=[
        pltpu.VMEM((3 * _QUAD,), jnp.int32),
        pltpu.VMEM((_BLK,), jnp.float32),
        pltpu.VMEM((2, _BLK), jnp.float32),
        pltpu.SemaphoreType.DMA,
        pltpu.SemaphoreType.DMA,
        pltpu.SemaphoreType.DMA,
    ],
    compiler_params=pltpu.CompilerParams(needs_layout_passes=False),
)
def _bank(w_hbm, idx_hbm, out_hbm, idx_v, in_v, out_v, sem0, sem_a, sem_b):
    wid = lax.axis_index("s") * 2 + lax.axis_index("c")
    pltpu.sync_copy(idx_hbm, idx_v)
    out_sems = (sem_a, sem_b)

    def per_oc(t, carry):
        oc = wid * _OC_PER_W + t
        pltpu.sync_copy(w_hbm.at[oc], in_v)
        c0 = pltpu.async_copy(in_v, out_hbm.at[oc, 0], sem0)
        copies = []
        for r in range(3):
            b = r & 1
            idx0 = tuple(
                idx_v[pl.ds(r * _QUAD + v * _LANES, _LANES)] for v in range(_VPQ))

            def body(i, idx, b=b):
                for v in range(_VPQ):
                    vals = plsc.load_gather(in_v, [idx[v]])
                    out_v[b, pl.ds(i + v * _LANES, _LANES)] = vals
                return tuple(x + _QUAD for x in idx)

            if r == 2:
                copies[0].wait()
            plsc.parallel_loop(0, _BLK, step=_QUAD, unroll=2, carry=idx0)(body)
            copies.append(
                pltpu.async_copy(out_v.at[b], out_hbm.at[oc, r + 1], out_sems[b]))
        c0.wait()
        copies[1].wait()
        copies[2].wait()
        return carry

    lax.fori_loop(0, _OC_PER_W, per_oc, 0)


def kernel(weight):
    w2 = weight.reshape(_OC, _BLK)
    out = _bank(w2, jnp.asarray(_IDX_TAB))
    return out.reshape(_OC, _ORD, _IC, _ORD, _K, _K)
